# Initial kernel scaffold; baseline (speedup 1.0000x reference)
import jax
import jax.numpy as jnp
from jax.experimental import pallas as pl


def kernel(x, edge_index):
    raise NotImplementedError("write your pallas kernel here")


# speedup vs baseline: 2.8598x; 2.8598x over previous
import functools

import jax
import jax.numpy as jnp
from jax import lax
from jax.experimental import pallas as pl
from jax.experimental.pallas import tpu as pltpu
from jax.experimental.pallas import tpu_sc as plsc

N_NODES_ = 10000
N_EDGES_ = 320000
D_ = 128

_info = plsc.get_sparse_core_info()
NC = _info.num_cores
NS = _info.num_subcores
NW = NC * NS

E_PER_W = N_EDGES_ // NW
CHUNK = 80
N_CHUNKS = E_PER_W // CHUNK


def _gather_body(idx_hbm, x_hbm, out_hbm, idx_v, rows_v, sem_i, sem_g, sem_o):
    wid = lax.axis_index("s") * NC + lax.axis_index("c")
    base = wid * E_PER_W

    def chunk(c, carry):
        off = base + c * CHUNK
        pltpu.sync_copy(idx_hbm.at[pl.ds(off, CHUNK)], idx_v)
        pltpu.async_copy(x_hbm.at[idx_v], rows_v, sem_g).wait()
        pltpu.sync_copy(rows_v, out_hbm.at[pl.ds(off, CHUNK)])
        return carry

    lax.fori_loop(0, N_CHUNKS, chunk, 0)


@jax.jit
def kernel(x, edge_index):
    idx = edge_index[1]
    mesh = plsc.VectorSubcoreMesh(core_axis_name="c", subcore_axis_name="s")
    run = pl.kernel(
        _gather_body,
        out_type=jax.ShapeDtypeStruct((N_EDGES_, D_), jnp.float32),
        mesh=mesh,
        scratch_types=[
            pltpu.VMEM((CHUNK,), jnp.int32),
            pltpu.VMEM((CHUNK, D_), jnp.float32),
            pltpu.SemaphoreType.DMA,
            pltpu.SemaphoreType.DMA,
            pltpu.SemaphoreType.DMA,
        ],
    )
    return run(idx, x)


# speedup vs baseline: 4.1524x; 1.4520x over previous
import functools

import jax
import jax.numpy as jnp
from jax import lax
from jax.experimental import pallas as pl
from jax.experimental.pallas import tpu as pltpu
from jax.experimental.pallas import tpu_sc as plsc

N_NODES_ = 10000
N_EDGES_ = 320000
D_ = 128

_info = plsc.get_sparse_core_info()
NC = _info.num_cores
NS = _info.num_subcores
NW = NC * NS

E_PER_W = N_EDGES_ // NW
CHUNK = 80
N_CHUNKS = E_PER_W // CHUNK
NBUF = 2


def _gather_body(idx_hbm, x_hbm, out_hbm,
                 idx0, idx1, rows0, rows1,
                 sem_i0, sem_i1, sem_g, sem_o0, sem_o1):
    idx_v = (idx0, idx1)
    rows_v = (rows0, rows1)
    sem_i = (sem_i0, sem_i1)
    sem_o = (sem_o0, sem_o1)
    wid = lax.axis_index("s") * NC + lax.axis_index("c")
    base = wid * E_PER_W

    def idx_copy(g, b):
        return pltpu.make_async_copy(
            idx_hbm.at[pl.ds(base + g * CHUNK, CHUNK)], idx_v[b], sem_i[b])

    def gather(b):
        return pltpu.make_async_copy(x_hbm.at[idx_v[b]], rows_v[b], sem_g)

    def store(g, b):
        return pltpu.make_async_copy(
            rows_v[b], out_hbm.at[pl.ds(base + g * CHUNK, CHUNK)], sem_o[b])

    for b in range(NBUF):
        idx_copy(b, b).start()

    def pair(i, carry):
        for b in range(NBUF):
            g = i * NBUF + b
            idx_copy(g, b).wait()

            @pl.when(i > 0)
            def _():
                store(g, b).wait()

            gather(b).start()
            gather(b).wait()

            @pl.when(g + NBUF < N_CHUNKS)
            def _():
                idx_copy(g + NBUF, b).start()

            store(g, b).start()
        return carry

    lax.fori_loop(0, N_CHUNKS // NBUF, pair, 0)

    g = N_CHUNKS - 1
    idx_copy(g, 0).wait()
    store(g, 0).wait()
    gather(0).start()
    gather(0).wait()
    store(g, 0).start()
    store(g, 0).wait()
    store(g - 1, 1).wait()


@jax.jit
def kernel(x, edge_index):
    idx = edge_index[1]
    mesh = plsc.VectorSubcoreMesh(core_axis_name="c", subcore_axis_name="s")
    run = pl.kernel(
        _gather_body,
        out_type=jax.ShapeDtypeStruct((N_EDGES_, D_), jnp.float32),
        mesh=mesh,
        scratch_types=[
            pltpu.VMEM((CHUNK,), jnp.int32),
            pltpu.VMEM((CHUNK,), jnp.int32),
            pltpu.VMEM((CHUNK, D_), jnp.float32),
            pltpu.VMEM((CHUNK, D_), jnp.float32),
            pltpu.SemaphoreType.DMA,
            pltpu.SemaphoreType.DMA,
            pltpu.SemaphoreType.DMA,
            pltpu.SemaphoreType.DMA,
            pltpu.SemaphoreType.DMA,
        ],
    )
    return run(idx, x)


# speedup vs baseline: 5.5427x; 1.3348x over previous
import functools

import jax
import jax.numpy as jnp
from jax import lax
from jax.experimental import pallas as pl
from jax.experimental.pallas import tpu as pltpu
from jax.experimental.pallas import tpu_sc as plsc

N_NODES_ = 10000
N_EDGES_ = 320000
D_ = 128

_info = plsc.get_sparse_core_info()
NC = _info.num_cores
NS = _info.num_subcores
NW = NC * NS

E_PER_W = N_EDGES_ // NW
CHUNK = 80
N_CHUNKS = E_PER_W // CHUNK
NBUF = 4


def _gather_body(idx_hbm, x_hbm, out_hbm, idx_v, rows_v, sem_i, sem_g, sem_o):
    wid = lax.axis_index("s") * NC + lax.axis_index("c")
    base = wid * E_PER_W

    def idx_copy(g, b):
        return pltpu.make_async_copy(
            idx_hbm.at[pl.ds(base + g * CHUNK, CHUNK)], idx_v[b], sem_i[b])

    def gather(b):
        return pltpu.make_async_copy(x_hbm.at[idx_v[b]], rows_v[b], sem_g[b])

    def store(g, b):
        return pltpu.make_async_copy(
            rows_v[b], out_hbm.at[pl.ds(base + g * CHUNK, CHUNK)], sem_o[b])

    for b in range(NBUF):
        idx_copy(b, b).start()

    def quad(i, carry):
        for b in range(NBUF):
            g = i * NBUF + b
            pb = (b - 1) % NBUF
            idx_copy(g, b).wait()

            @pl.when(i > 0)
            def _():
                store(g, b).wait()

            gather(b).start()

            @pl.when(g > 0)
            def _():
                gather(pb).wait()

                @pl.when(g + NBUF - 1 < N_CHUNKS)
                def _():
                    idx_copy(g + NBUF - 1, pb).start()

                store(g - 1, pb).start()
        return carry

    lax.fori_loop(0, (N_CHUNKS - 1) // NBUF, quad, 0)

    last = N_CHUNKS - 1
    gather((last - 1) % NBUF).wait()
    store(last - 1, (last - 1) % NBUF).start()
    idx_copy(last, 0).wait()
    store(last, 0).wait()
    gather(0).start()
    gather(0).wait()
    store(last, 0).start()
    for b in range(NBUF):
        store(last - ((last % NBUF) - b) % NBUF, b).wait()


@jax.jit
def kernel(x, edge_index):
    idx = edge_index[1]
    mesh = plsc.VectorSubcoreMesh(core_axis_name="c", subcore_axis_name="s")
    run = pl.kernel(
        _gather_body,
        out_type=jax.ShapeDtypeStruct((N_EDGES_, D_), jnp.float32),
        mesh=mesh,
        scratch_types=[
            tuple(pltpu.VMEM((CHUNK,), jnp.int32) for _ in range(NBUF)),
            tuple(pltpu.VMEM((CHUNK, D_), jnp.float32) for _ in range(NBUF)),
            tuple(pltpu.SemaphoreType.DMA for _ in range(NBUF)),
            tuple(pltpu.SemaphoreType.DMA for _ in range(NBUF)),
            tuple(pltpu.SemaphoreType.DMA for _ in range(NBUF)),
        ],
    )
    return run(idx, x)


# speedup vs baseline: 5.7645x; 1.0400x over previous
import functools

import jax
import jax.numpy as jnp
from jax import lax
from jax.experimental import pallas as pl
from jax.experimental.pallas import tpu as pltpu
from jax.experimental.pallas import tpu_sc as plsc

N_NODES_ = 10000
N_EDGES_ = 320000
D_ = 128

_info = plsc.get_sparse_core_info()
NC = _info.num_cores
NS = _info.num_subcores
NW = NC * NS

E_PER_W = N_EDGES_ // NW
CHUNK = 200
N_CHUNKS = E_PER_W // CHUNK
NBUF = 4


def _gather_body(idx_hbm, x_hbm, out_hbm, idx_v, rows_v, sem_i, sem_g, sem_o):
    wid = lax.axis_index("s") * NC + lax.axis_index("c")
    base = wid * E_PER_W

    def idx_copy(g, b):
        return pltpu.make_async_copy(
            idx_hbm.at[pl.ds(base + g * CHUNK, CHUNK)], idx_v[b], sem_i[b])

    def gather(b):
        return pltpu.make_async_copy(x_hbm.at[idx_v[b]], rows_v[b], sem_g[b])

    def store(g, b):
        return pltpu.make_async_copy(
            rows_v[b], out_hbm.at[pl.ds(base + g * CHUNK, CHUNK)], sem_o[b])

    for b in range(NBUF):
        idx_copy(b, b).start()

    def quad(i, carry):
        for b in range(NBUF):
            g = i * NBUF + b
            pb = (b - 1) % NBUF
            idx_copy(g, b).wait()

            @pl.when(i > 0)
            def _():
                store(g, b).wait()

            gather(b).start()

            @pl.when(g > 0)
            def _():
                gather(pb).wait()

                @pl.when(g + NBUF - 1 < N_CHUNKS)
                def _():
                    idx_copy(g + NBUF - 1, pb).start()

                store(g - 1, pb).start()
        return carry

    nquad = (N_CHUNKS - 1) // NBUF
    lax.fori_loop(0, nquad, quad, 0)

    for g in range(nquad * NBUF, N_CHUNKS - 1):
        b = g % NBUF
        pb = (b - 1) % NBUF
        idx_copy(g, b).wait()
        if g >= NBUF:
            store(g, b).wait()
        gather(b).start()
        if g > 0:
            gather(pb).wait()
            if g + NBUF - 1 < N_CHUNKS:
                idx_copy(g + NBUF - 1, pb).start()
            store(g - 1, pb).start()

    last = N_CHUNKS - 1
    bl = last % NBUF
    gather((last - 1) % NBUF).wait()
    store(last - 1, (last - 1) % NBUF).start()
    idx_copy(last, bl).wait()
    store(last, bl).wait()
    gather(bl).start()
    gather(bl).wait()
    store(last, bl).start()
    for b in range(NBUF):
        store(last - ((last - b) % NBUF), b).wait()


@jax.jit
def kernel(x, edge_index):
    idx = edge_index[1]
    mesh = plsc.VectorSubcoreMesh(core_axis_name="c", subcore_axis_name="s")
    run = pl.kernel(
        _gather_body,
        out_type=jax.ShapeDtypeStruct((N_EDGES_, D_), jnp.float32),
        mesh=mesh,
        scratch_types=[
            tuple(pltpu.VMEM((CHUNK,), jnp.int32) for _ in range(NBUF)),
            tuple(pltpu.VMEM((CHUNK, D_), jnp.float32) for _ in range(NBUF)),
            tuple(pltpu.SemaphoreType.DMA for _ in range(NBUF)),
            tuple(pltpu.SemaphoreType.DMA for _ in range(NBUF)),
            tuple(pltpu.SemaphoreType.DMA for _ in range(NBUF)),
        ],
    )
    return run(idx, x)
